# baseline (speedup 1.0000x reference)
import functools

import jax
import jax.numpy as jnp
import numpy as np
from jax import lax
from jax.experimental import pallas as pl
from jax.experimental.pallas import tpu as pltpu
from jax.experimental.pallas import tpu_sc as plsc

N = 10000
E = 320000
LAT = 128
STEPS = 3
R_MAX = 5.0
NB = 8

NC, NS = 2, 16
NW = NC * NS
K = 80
NJ = 128
ECH = NJ * K
EPAD = NW * ECH
EP = EPAD // 128
PBLK = 160
NROW = 624
NTAIL = N - NS * NROW

NH = 4
EH = EPAD // NH
ECH2 = ECH // NH
NJ2 = NJ // NH
EP2 = EP // NH

EBLK = 4096
EG = EPAD // EBLK
NBLK = 2000
NG = N // NBLK

_SC_MESH = plsc.VectorSubcoreMesh(core_axis_name="c", subcore_axis_name="s")

f32 = jnp.float32
i32 = jnp.int32
bf16 = jnp.bfloat16


def _mm(a, b):
    return lax.dot_general(a, b, (((1,), (0,)), ((), ())),
                           preferred_element_type=f32)


def _mmT(a, b):
    return lax.dot_general(a, b, (((0,), (0,)), ((), ())),
                           preferred_element_type=f32)


def _rms(x):
    return x * lax.rsqrt(jnp.mean(x * x, axis=-1, keepdims=True) + 1e-6)


def _silu(x):
    return x * jax.nn.sigmoid(x)


def _vec_body(pos_hbm, snd_hbm, rcv_hbm, out_hbm, pos_v, snd_v, rcv_v, obuf):
    wid = lax.axis_index("s") * NC + lax.axis_index("c")
    pltpu.sync_copy(pos_hbm, pos_v)
    pltpu.sync_copy(snd_hbm.at[pl.ds(wid * ECH, ECH)], snd_v)
    pltpu.sync_copy(rcv_hbm.at[pl.ds(wid * ECH, ECH)], rcv_v)

    def body(i, carry):
        s4 = snd_v[pl.ds(i * 16, 16)] * 4
        r4 = rcv_v[pl.ds(i * 16, 16)] * 4
        l16 = lax.iota(i32, 16) + i * 16
        for c in range(3):
            ps = plsc.load_gather(pos_v, [s4 + c])
            pr = plsc.load_gather(pos_v, [r4 + c])
            plsc.store_scatter(obuf, [l16 + c * ECH], pr - ps)
        return carry

    lax.fori_loop(0, ECH // 16, body, 0)
    for c in range(3):
        pltpu.sync_copy(obuf.at[pl.ds(c * ECH, ECH)],
                        out_hbm.at[pl.ds(c * EPAD + wid * ECH, ECH)])


_vec_call = functools.partial(
    pl.kernel,
    out_type=jax.ShapeDtypeStruct((4 * EPAD,), f32),
    mesh=_SC_MESH,
    compiler_params=pltpu.CompilerParams(needs_layout_passes=False),
    scratch_types=[
        pltpu.VMEM((N * 4,), f32),
        pltpu.VMEM((ECH,), i32),
        pltpu.VMEM((ECH,), i32),
        pltpu.VMEM((3 * ECH,), f32),
    ],
)(_vec_body)


_NBUF = 4


_LAG = 2


def _gath_body(ech, nj, hs_hbm, hr_hbm, snd_hbm, rcv_hbm, gs_hbm, gr_hbm,
               snd_v, rcv_v,
               a0, a1, a2, a3, b0, b1, b2, b3,
               sa0, sa1, sa2, sa3, sb0, sb1, sb2, sb3,
               wa0, wa1, wa2, wa3, wb0, wb1, wb2, wb3):
    wid = lax.axis_index("s") * NC + lax.axis_index("c")
    pltpu.sync_copy(snd_hbm.at[pl.ds(wid * ech, ech)], snd_v)
    pltpu.sync_copy(rcv_hbm.at[pl.ds(wid * ech, ech)], rcv_v)
    bufa, bufb = [a0, a1, a2, a3], [b0, b1, b2, b3]
    sema, semb = [sa0, sa1, sa2, sa3], [sb0, sb1, sb2, sb3]
    wema, wemb = [wa0, wa1, wa2, wa3], [wb0, wb1, wb2, wb3]

    def issue_g(j, b):
        pltpu.async_copy(hs_hbm.at[snd_v.at[pl.ds(j * K, K)]],
                         bufa[b], sema[b])
        pltpu.async_copy(hr_hbm.at[rcv_v.at[pl.ds(j * K, K)]],
                         bufb[b], semb[b])

    def wait_g(b):
        pltpu.make_async_copy(hs_hbm.at[pl.ds(0, K)], bufa[b],
                              sema[b]).wait()
        pltpu.make_async_copy(hr_hbm.at[pl.ds(0, K)], bufb[b],
                              semb[b]).wait()

    def issue_w(j, b):
        base = wid * ech + j * K
        pltpu.async_copy(bufa[b], gs_hbm.at[pl.ds(base, K)], wema[b])
        pltpu.async_copy(bufb[b], gr_hbm.at[pl.ds(base, K)], wemb[b])

    def wait_w(b):
        pltpu.make_async_copy(bufa[b], gs_hbm.at[pl.ds(0, K)],
                              wema[b]).wait()
        pltpu.make_async_copy(bufb[b], gr_hbm.at[pl.ds(0, K)],
                              wemb[b]).wait()

    def body(jj, carry):
        for b in range(_NBUF):
            j = jj * _NBUF + b

            @pl.when(jj > 0)
            def _():
                wait_w(b)

            issue_g(j, b)
            bd = (b - _LAG) % _NBUF

            @pl.when(j >= _LAG)
            def _():
                wait_g(bd)
                issue_w(j - _LAG, bd)

        return carry

    lax.fori_loop(0, nj // _NBUF, body, 0)
    for t in range(_LAG):
        j = nj - _LAG + t
        wait_g(j % _NBUF)
        issue_w(j, j % _NBUF)
    for b in range(_NBUF):
        wait_w(b)


def _make_gath(ech, nj, rows):
    return functools.partial(
        pl.kernel,
        out_type=(jax.ShapeDtypeStruct((rows, LAT), f32),
                  jax.ShapeDtypeStruct((rows, LAT), f32)),
        mesh=_SC_MESH,
        scratch_types=[
            pltpu.VMEM((ech,), i32),
            pltpu.VMEM((ech,), i32),
        ] + [pltpu.VMEM((K, LAT), f32)] * (2 * _NBUF)
          + [pltpu.SemaphoreType.DMA] * (4 * _NBUF),
    )(functools.partial(_gath_body, ech, nj))


_gath_call = _make_gath(ECH2, NJ2, EH)


def _scat_body(ech, nj, msg_hbm, rcv3_hbm, zero_hbm, out_hbm, shared,
               m0, m1, rbuf, sl0, sl1):
    cid = lax.axis_index("c")
    sid = lax.axis_index("s")
    wid = sid * NC + cid
    pltpu.sync_copy(rcv3_hbm.at[wid], rbuf)
    pltpu.sync_copy(zero_hbm.at[pl.ds(sid * NROW, NROW)],
                    shared.at[pl.ds(sid * NROW, NROW)])

    @pl.when(sid == NS - 1)
    def _():
        pltpu.sync_copy(zero_hbm.at[pl.ds(NS * NROW, NTAIL)],
                        shared.at[pl.ds(NS * NROW, NTAIL)])

    plsc.subcore_barrier()
    bufm = [m0, m1]
    seml = [sl0, sl1]

    def issue(j, b):
        pltpu.async_copy(msg_hbm.at[pl.ds(wid * ech + j * K, K)],
                         bufm[b], seml[b])

    def drain(j, b):
        pltpu.make_async_copy(msg_hbm.at[pl.ds(0, K)], bufm[b],
                              seml[b]).wait()
        pltpu.sync_copy(bufm[b], shared.at[rbuf.at[j]], add=True)

    issue(0, 0)
    issue(1, 1)

    def body(jj, carry):
        for b in range(2):
            j = jj * 2 + b
            drain(j, b)
            issue(j + 2, b)
        return carry

    lax.fori_loop(0, nj // 2 - 1, body, 0)
    for b in range(2):
        drain(nj - 2 + b, b)
    plsc.subcore_barrier()
    pltpu.sync_copy(shared.at[pl.ds(sid * NROW, NROW)],
                    out_hbm.at[cid, pl.ds(sid * NROW, NROW)])

    @pl.when(sid == NS - 1)
    def _():
        pltpu.sync_copy(shared.at[pl.ds(NS * NROW, NTAIL)],
                        out_hbm.at[cid, pl.ds(NS * NROW, NTAIL)])


def _make_scat(ech, nj):
    return functools.partial(
        pl.kernel,
        out_type=jax.ShapeDtypeStruct((NC, N, LAT), f32),
        mesh=_SC_MESH,
        scratch_types=[
            pltpu.VMEM_SHARED((N, LAT), f32),
            pltpu.VMEM((K, LAT), f32),
            pltpu.VMEM((K, LAT), f32),
            pltpu.VMEM((nj, K), i32),
            pltpu.SemaphoreType.DMA,
            pltpu.SemaphoreType.DMA,
        ],
    )(functools.partial(_scat_body, ech, nj))


_scat_call = _make_scat(ECH2, NJ2)


def _geom_body(boff, vec_ref, bm_ref, bemb_ref, cut_ref, feat_ref, env_ref):
    v = vec_ref[...]
    vx, vy, vz = v[0], v[1], v[2]
    r = jnp.sqrt(vx * vx + vy * vy + vz * vz + 1e-12)
    inv = 1.0 / (r + 1e-9)
    ux, uy, uz = vx * inv, vy * inv, vz * inv
    pref = np.float32(np.sqrt(2.0 / R_MAX))
    cols = [pref * jnp.sin(np.float32(n * np.pi / R_MAX) * r) * inv
            for n in range(1, NB + 1)]
    s3 = np.float32(np.sqrt(3.0))
    s5 = np.float32(np.sqrt(5.0))
    s15 = np.float32(np.sqrt(15.0))
    cols += [jnp.ones_like(ux), s3 * ux, s3 * uy, s3 * uz,
             s15 * ux * uy, s15 * uy * uz,
             np.float32(0.5) * s5 * (3.0 * uz * uz - 1.0),
             s15 * ux * uz,
             np.float32(0.5) * s15 * (ux * ux - uy * uy)]
    m = bm_ref[...].astype(f32)
    for j in range(16):
        b0 = bemb_ref[0:1, j:j + 1]
        b1 = bemb_ref[1:2, j:j + 1]
        cols.append((1.0 - m) * b0 + m * b1)
    zero = jnp.zeros_like(r)
    cols += [zero] * 7
    feat_ref[...] = jnp.stack(cols, axis=0)
    cut = cut_ref[...]
    x = jnp.clip(r / cut, 0.0, 1.0)
    env = 0.5 * (jnp.cos(np.float32(np.pi) * x) + 1.0) * (r < cut)
    gid = (lax.broadcasted_iota(i32, (PBLK, 128), 0)
           + (pl.program_id(0) + boff) * PBLK) * 128 \
        + lax.broadcasted_iota(i32, (PBLK, 128), 1)
    env_ref[...] = jnp.where(gid < E, env, 0.0)


def _geom_call(vecp, bmp, bemb, cut, t):
    boff = t * (EP2 // PBLK)
    return pl.pallas_call(
        functools.partial(_geom_body, boff),
        grid=(EP2 // PBLK,),
        in_specs=[pl.BlockSpec((4, PBLK, 128),
                               lambda i: (0, i + boff, 0)),
                  pl.BlockSpec((PBLK, 128), lambda i: (i + boff, 0)),
                  pl.BlockSpec((2, 16), lambda i: (0, 0)),
                  pl.BlockSpec((1, 1), lambda i: (0, 0))],
        out_specs=[pl.BlockSpec((40, PBLK, 128), lambda i: (0, i, 0)),
                   pl.BlockSpec((PBLK, 128), lambda i: (i, 0))],
        out_shape=[jax.ShapeDtypeStruct((40, EP2, 128), f32),
                   jax.ShapeDtypeStruct((EP2, 128), f32)],
    )(vecp, bmp, bemb, cut)


def _edge_mlp_body(ft_ref, w1_ref, b1_ref, w2_ref, b2_ref, e_ref):
    h1 = _silu(_mmT(ft_ref[...], w1_ref[...]) + b1_ref[...])
    e_ref[...] = _rms(_mm(h1, w2_ref[...]) + b2_ref[...])


def _node_body(ati_ref, aci_ref, rci_ref, cn_ref, ta_ref, tb_ref, tc_ref,
               w1_ref, b1_ref, w2_ref, b2_ref, ws_ref, wr_ref,
               h_ref, hs_ref, hr_ref):
    i32w = lax.broadcasted_iota(i32, (1, 32), 1)
    i16w = lax.broadcasted_iota(i32, (1, 16), 1)
    emb_a = _mm((ati_ref[...] == i32w).astype(f32), ta_ref[...])
    emb_b = _mm((aci_ref[...] == i16w).astype(f32), tb_ref[...])
    emb_c = _mm((rci_ref[...] == i32w).astype(f32), tc_ref[...])
    x = jnp.concatenate(
        [emb_a, emb_b, emb_c, cn_ref[...],
         jnp.zeros((emb_a.shape[0], 7), f32)], axis=1)
    h1 = _silu(_mm(x, w1_ref[...]) + b1_ref[...])
    h = _rms(_mm(h1, w2_ref[...]) + b2_ref[...])
    h_ref[...] = h
    hs_ref[...] = _mm(h, ws_ref[...])
    hr_ref[...] = _mm(h, wr_ref[...])


def _mmb(a, b):
    return lax.dot_general(a.astype(jnp.bfloat16), b.astype(jnp.bfloat16),
                           (((1,), (0,)), ((), ())),
                           preferred_element_type=f32)


def _msg_body(e_ref, gs_ref, gr_ref, env_ref, w1e_ref, b1_ref, w2m_ref,
              b2m_ref, w2g_ref, b2g_ref, enew_ref, msg_ref):
    e = e_ref[...]
    pre = (_mmb(e, w1e_ref[...]) + gs_ref[...] + gr_ref[...]
           + b1_ref[...])
    s1 = _silu(pre)
    o1 = _mmb(s1, w2m_ref[...]) + b2m_ref[...]
    og = jnp.sum(s1 * w2g_ref[...], axis=-1, keepdims=True) + b2g_ref[...]
    msg = _rms(o1) * jax.nn.sigmoid(og) * env_ref[...]
    enew_ref[...] = e + msg
    msg_ref[...] = msg


def _upd_body(na, h_ref, *refs):
    aggs = refs[:na]
    (w1h_ref, w1a_ref, b1_ref, w2_ref, b2_ref, ws_ref, wr_ref,
     out_ref, hs_ref, hr_ref) = refs[na:]
    h = h_ref[...]
    a = aggs[0][...]
    for r in aggs[1:]:
        a = a + r[...]
    s1 = _silu(_mm(h, w1h_ref[...]) + _mm(a, w1a_ref[...]) + b1_ref[...])
    hn = h + _rms(_mm(s1, w2_ref[...]) + b2_ref[...])
    out_ref[...] = hn
    hs_ref[...] = _mm(hn, ws_ref[...])
    hr_ref[...] = _mm(hn, wr_ref[...])


def _head_body(h_ref, w1_ref, b1_ref, w2_ref, b2_ref, out_ref):
    s1 = _silu(_mm(h_ref[...], w1_ref[...]) + b1_ref[...])
    out_ref[...] = _mm(s1, w2_ref[...]) + b2_ref[...]


def _full(shape):
    return pl.BlockSpec(shape, lambda i: (0,) * len(shape))


def _rows(blk, width):
    return pl.BlockSpec((blk, width), lambda i: (i, 0))


def _edge_mlp_call(ft, w1, b1, w2, b2):
    rows = ft.shape[1]
    return pl.pallas_call(
        _edge_mlp_body,
        grid=(rows // EBLK,),
        in_specs=[pl.BlockSpec((40, EBLK), lambda i: (0, i)),
                  _full((40, LAT)), _full((1, LAT)), _full((LAT, LAT)),
                  _full((1, LAT))],
        out_specs=_rows(EBLK, LAT),
        out_shape=jax.ShapeDtypeStruct((rows, LAT), f32),
    )(ft, w1, b1, w2, b2)


def _node_call(ati, aci, rci, cn, ta, tb, tc, w1, b1, w2, b2, ws, wr):
    return pl.pallas_call(
        _node_body,
        grid=(NG,),
        in_specs=[_rows(NBLK, 1), _rows(NBLK, 1), _rows(NBLK, 1),
                  _rows(NBLK, 1), _full((32, 32)), _full((16, 16)),
                  _full((32, 16)), _full((72, LAT)), _full((1, LAT)),
                  _full((LAT, LAT)), _full((1, LAT)),
                  _full((LAT, LAT)), _full((LAT, LAT))],
        out_specs=[_rows(NBLK, LAT)] * 3,
        out_shape=[jax.ShapeDtypeStruct((N, LAT), f32)] * 3,
    )(ati, aci, rci, cn, ta, tb, tc, w1, b1, w2, b2, ws, wr)


def _msg_call(e, gs, gr, env, w1e, b1, w2m, b2m, w2g, b2g):
    rows = e.shape[0]
    return pl.pallas_call(
        _msg_body,
        grid=(rows // EBLK,),
        in_specs=[_rows(EBLK, LAT), _rows(EBLK, LAT), _rows(EBLK, LAT),
                  _rows(EBLK, 1), _full((LAT, LAT)), _full((1, LAT)),
                  _full((LAT, LAT)), _full((1, LAT)), _full((1, LAT)),
                  _full((1, 1))],
        out_specs=[_rows(EBLK, LAT), _rows(EBLK, LAT)],
        out_shape=[jax.ShapeDtypeStruct((rows, LAT), f32),
                   jax.ShapeDtypeStruct((rows, LAT), f32)],
    )(e, gs, gr, env, w1e, b1, w2m, b2m, w2g, b2g)


def _upd_call(h, aggs, w1h, w1a, b1, w2, b2, ws, wr):
    na = len(aggs)
    return pl.pallas_call(
        functools.partial(_upd_body, na),
        grid=(NG,),
        in_specs=[_rows(NBLK, LAT)] * (1 + na) +
                 [_full((LAT, LAT)), _full((LAT, LAT)), _full((1, LAT)),
                  _full((LAT, LAT)), _full((1, LAT)),
                  _full((LAT, LAT)), _full((LAT, LAT))],
        out_specs=[_rows(NBLK, LAT)] * 3,
        out_shape=[jax.ShapeDtypeStruct((N, LAT), f32)] * 3,
    )(h, *aggs, w1h, w1a, b1, w2, b2, ws, wr)


def _head_call(h, w1, b1, w2, b2):
    return pl.pallas_call(
        _head_body,
        grid=(NG,),
        in_specs=[_rows(NBLK, LAT), _full((LAT, LAT)), _full((1, LAT)),
                  _full((LAT, 8)), _full((1, 8))],
        out_specs=_rows(NBLK, 8),
        out_shape=jax.ShapeDtypeStruct((N, 8), f32),
    )(h, w1, b1, w2, b2)


def kernel(pos, c_noise, atom_type_emb, atom_code_emb, residue_code_emb,
           bond_emb, node_W1, node_b1, node_W2, node_b2, edge_W1, edge_b1,
           edge_W2, edge_b2, msg_W1, msg_b1, msg_W2, msg_b2, upd_W1, upd_b1,
           upd_W2, upd_b2, head_W1, head_b1, head_W2, head_b2,
           atom_type_index, atom_code_index, residue_code_index,
           residue_sequence_index, bond_mask, edge_index,
           effective_radial_cutoff):
    snd = jnp.pad(edge_index[0].astype(i32), (0, EPAD - E))
    rcv = jnp.pad(edge_index[1].astype(i32), (0, EPAD - E))
    cut = jnp.asarray(effective_radial_cutoff, f32).reshape(1, 1)

    pos4 = jnp.pad(pos, ((0, 0), (0, 1))).reshape(-1)
    vecp = _vec_call(pos4, snd, rcv).reshape(4, EP, 128)

    bmp = jnp.pad(bond_mask.astype(i32), (0, EPAD - E)).reshape(EP, 128)
    w1p = jnp.pad(edge_W1, ((0, 7), (0, 0)))
    env, e = [], []
    for t in range(NH):
        feat_t, envp_t = _geom_call(vecp, bmp, bond_emb, cut, t)
        env.append(envp_t.reshape(EH, 1))
        e.append(_edge_mlp_call(feat_t.reshape(40, EH), w1p,
                                edge_b1.reshape(1, LAT), edge_W2,
                                edge_b2.reshape(1, LAT)))

    h, hs, hr = _node_call(
        atom_type_index.reshape(N, 1).astype(i32),
        atom_code_index.reshape(N, 1).astype(i32),
        residue_code_index.reshape(N, 1).astype(i32),
        c_noise.reshape(N, 1),
        jnp.pad(atom_type_emb, ((0, 12), (0, 0))),
        jnp.pad(atom_code_emb, ((0, 6), (0, 0))),
        jnp.pad(residue_code_emb, ((0, 7), (0, 0))),
        jnp.pad(node_W1, ((0, 7), (0, 0))), node_b1.reshape(1, LAT),
        node_W2, node_b2.reshape(1, LAT),
        msg_W1[0][LAT:2 * LAT], msg_W1[0][2 * LAT:])

    snd_h = [snd[t * EH:(t + 1) * EH] for t in range(NH)]
    rcv_h = [rcv[t * EH:(t + 1) * EH] for t in range(NH)]
    rcv3_h = [r.reshape(NW, NJ2, K) for r in rcv_h]
    zero_h = jnp.zeros((N, LAT), f32)

    for s in range(STEPS):
        parts = []
        for t in range(NH):
            gs, gr = _gath_call(hs, hr, snd_h[t], rcv_h[t])
            e[t], msg = _msg_call(
                e[t], gs, gr, env[t], msg_W1[s][:LAT],
                msg_b1[s].reshape(1, LAT),
                msg_W2[s][:, :LAT], msg_b2[s][:LAT].reshape(1, LAT),
                msg_W2[s][:, LAT:].reshape(1, LAT),
                msg_b2[s][LAT:].reshape(1, 1))
            parts.append(_scat_call(msg, rcv3_h[t], zero_h))
        sn = min(s + 1, STEPS - 1)
        aggs = [p[c] for p in parts for c in range(NC)]
        h, hs, hr = _upd_call(
            h, aggs, upd_W1[s][:LAT], upd_W1[s][LAT:],
            upd_b1[s].reshape(1, LAT), upd_W2[s], upd_b2[s].reshape(1, LAT),
            msg_W1[sn][LAT:2 * LAT], msg_W1[sn][2 * LAT:])

    pred = _head_call(
        h, head_W1, head_b1.reshape(1, LAT),
        jnp.pad(head_W2, ((0, 0), (0, 5))),
        jnp.pad(head_b2, (0, 5)).reshape(1, 8))
    return pred[:, :3]


# speedup vs baseline: 1.0307x; 1.0307x over previous
import functools

import jax
import jax.numpy as jnp
import numpy as np
from jax import lax
from jax.experimental import pallas as pl
from jax.experimental.pallas import tpu as pltpu
from jax.experimental.pallas import tpu_sc as plsc

N = 10000
E = 320000
LAT = 128
STEPS = 3
R_MAX = 5.0
NB = 8

NC, NS = 2, 16
NW = NC * NS
K = 80
NJ = 128
ECH = NJ * K
EPAD = NW * ECH
EP = EPAD // 128
PBLK = 256
NROW = 624
NTAIL = N - NS * NROW

NH = 2
EH = EPAD // NH
ECH2 = ECH // NH
NJ2 = NJ // NH
EP2 = EP // NH

EBLK = 4096
EG = EPAD // EBLK
NBLK = 2000
NG = N // NBLK

_SC_MESH = plsc.VectorSubcoreMesh(core_axis_name="c", subcore_axis_name="s")

f32 = jnp.float32
i32 = jnp.int32
bf16 = jnp.bfloat16


def _mm(a, b):
    return lax.dot_general(a, b, (((1,), (0,)), ((), ())),
                           preferred_element_type=f32)


def _mmT(a, b):
    return lax.dot_general(a, b, (((0,), (0,)), ((), ())),
                           preferred_element_type=f32)


def _rms(x):
    return x * lax.rsqrt(jnp.mean(x * x, axis=-1, keepdims=True) + 1e-6)


def _silu(x):
    return x * jax.nn.sigmoid(x)


def _vec_body(pos_hbm, snd_hbm, rcv_hbm, out_hbm, pos_v, snd_v, rcv_v, obuf):
    wid = lax.axis_index("s") * NC + lax.axis_index("c")
    pltpu.sync_copy(pos_hbm, pos_v)
    pltpu.sync_copy(snd_hbm.at[pl.ds(wid * ECH, ECH)], snd_v)
    pltpu.sync_copy(rcv_hbm.at[pl.ds(wid * ECH, ECH)], rcv_v)

    def body(i, carry):
        s4 = snd_v[pl.ds(i * 16, 16)] * 4
        r4 = rcv_v[pl.ds(i * 16, 16)] * 4
        l16 = lax.iota(i32, 16) + i * 16
        for c in range(3):
            ps = plsc.load_gather(pos_v, [s4 + c])
            pr = plsc.load_gather(pos_v, [r4 + c])
            plsc.store_scatter(obuf, [l16 + c * ECH], pr - ps)
        return carry

    lax.fori_loop(0, ECH // 16, body, 0)
    for c in range(3):
        pltpu.sync_copy(obuf.at[pl.ds(c * ECH, ECH)],
                        out_hbm.at[pl.ds(c * EPAD + wid * ECH, ECH)])


_vec_call = functools.partial(
    pl.kernel,
    out_type=jax.ShapeDtypeStruct((4 * EPAD,), f32),
    mesh=_SC_MESH,
    compiler_params=pltpu.CompilerParams(needs_layout_passes=False),
    scratch_types=[
        pltpu.VMEM((N * 4,), f32),
        pltpu.VMEM((ECH,), i32),
        pltpu.VMEM((ECH,), i32),
        pltpu.VMEM((3 * ECH,), f32),
    ],
)(_vec_body)


_NBUF = 4


_LAG = 2


def _gath_body(ech, nj, hs_hbm, hr_hbm, snd_hbm, rcv_hbm, gs_hbm, gr_hbm,
               snd_v, rcv_v,
               a0, a1, a2, a3, b0, b1, b2, b3,
               sa0, sa1, sa2, sa3, sb0, sb1, sb2, sb3,
               wa0, wa1, wa2, wa3, wb0, wb1, wb2, wb3):
    wid = lax.axis_index("s") * NC + lax.axis_index("c")
    pltpu.sync_copy(snd_hbm.at[pl.ds(wid * ech, ech)], snd_v)
    pltpu.sync_copy(rcv_hbm.at[pl.ds(wid * ech, ech)], rcv_v)
    bufa, bufb = [a0, a1, a2, a3], [b0, b1, b2, b3]
    sema, semb = [sa0, sa1, sa2, sa3], [sb0, sb1, sb2, sb3]
    wema, wemb = [wa0, wa1, wa2, wa3], [wb0, wb1, wb2, wb3]

    def issue_g(j, b):
        pltpu.async_copy(hs_hbm.at[snd_v.at[pl.ds(j * K, K)]],
                         bufa[b], sema[b])
        pltpu.async_copy(hr_hbm.at[rcv_v.at[pl.ds(j * K, K)]],
                         bufb[b], semb[b])

    def wait_g(b):
        pltpu.make_async_copy(hs_hbm.at[pl.ds(0, K)], bufa[b],
                              sema[b]).wait()
        pltpu.make_async_copy(hr_hbm.at[pl.ds(0, K)], bufb[b],
                              semb[b]).wait()

    def issue_w(j, b):
        base = wid * ech + j * K
        pltpu.async_copy(bufa[b], gs_hbm.at[pl.ds(base, K)], wema[b])
        pltpu.async_copy(bufb[b], gr_hbm.at[pl.ds(base, K)], wemb[b])

    def wait_w(b):
        pltpu.make_async_copy(bufa[b], gs_hbm.at[pl.ds(0, K)],
                              wema[b]).wait()
        pltpu.make_async_copy(bufb[b], gr_hbm.at[pl.ds(0, K)],
                              wemb[b]).wait()

    def body(jj, carry):
        for b in range(_NBUF):
            j = jj * _NBUF + b

            @pl.when(jj > 0)
            def _():
                wait_w(b)

            issue_g(j, b)
            bd = (b - _LAG) % _NBUF

            @pl.when(j >= _LAG)
            def _():
                wait_g(bd)
                issue_w(j - _LAG, bd)

        return carry

    lax.fori_loop(0, nj // _NBUF, body, 0)
    for t in range(_LAG):
        j = nj - _LAG + t
        wait_g(j % _NBUF)
        issue_w(j, j % _NBUF)
    for b in range(_NBUF):
        wait_w(b)


def _make_gath(ech, nj, rows):
    return functools.partial(
        pl.kernel,
        out_type=(jax.ShapeDtypeStruct((rows, LAT), f32),
                  jax.ShapeDtypeStruct((rows, LAT), f32)),
        mesh=_SC_MESH,
        scratch_types=[
            pltpu.VMEM((ech,), i32),
            pltpu.VMEM((ech,), i32),
        ] + [pltpu.VMEM((K, LAT), f32)] * (2 * _NBUF)
          + [pltpu.SemaphoreType.DMA] * (4 * _NBUF),
    )(functools.partial(_gath_body, ech, nj))


_gath_call = _make_gath(ECH2, NJ2, EH)


def _scat_body(ech, nj, msg_hbm, rcv3_hbm, zero_hbm, out_hbm, shared,
               m0, m1, rbuf, sl0, sl1):
    cid = lax.axis_index("c")
    sid = lax.axis_index("s")
    wid = sid * NC + cid
    pltpu.sync_copy(rcv3_hbm.at[wid], rbuf)
    pltpu.sync_copy(zero_hbm.at[pl.ds(sid * NROW, NROW)],
                    shared.at[pl.ds(sid * NROW, NROW)])

    @pl.when(sid == NS - 1)
    def _():
        pltpu.sync_copy(zero_hbm.at[pl.ds(NS * NROW, NTAIL)],
                        shared.at[pl.ds(NS * NROW, NTAIL)])

    plsc.subcore_barrier()
    bufm = [m0, m1]
    seml = [sl0, sl1]

    def issue(j, b):
        pltpu.async_copy(msg_hbm.at[pl.ds(wid * ech + j * K, K)],
                         bufm[b], seml[b])

    def drain(j, b):
        pltpu.make_async_copy(msg_hbm.at[pl.ds(0, K)], bufm[b],
                              seml[b]).wait()
        pltpu.sync_copy(bufm[b], shared.at[rbuf.at[j]], add=True)

    issue(0, 0)
    issue(1, 1)

    def body(jj, carry):
        for b in range(2):
            j = jj * 2 + b
            drain(j, b)
            issue(j + 2, b)
        return carry

    lax.fori_loop(0, nj // 2 - 1, body, 0)
    for b in range(2):
        drain(nj - 2 + b, b)
    plsc.subcore_barrier()
    pltpu.sync_copy(shared.at[pl.ds(sid * NROW, NROW)],
                    out_hbm.at[cid, pl.ds(sid * NROW, NROW)])

    @pl.when(sid == NS - 1)
    def _():
        pltpu.sync_copy(shared.at[pl.ds(NS * NROW, NTAIL)],
                        out_hbm.at[cid, pl.ds(NS * NROW, NTAIL)])


def _make_scat(ech, nj):
    return functools.partial(
        pl.kernel,
        out_type=jax.ShapeDtypeStruct((NC, N, LAT), f32),
        mesh=_SC_MESH,
        scratch_types=[
            pltpu.VMEM_SHARED((N, LAT), f32),
            pltpu.VMEM((K, LAT), f32),
            pltpu.VMEM((K, LAT), f32),
            pltpu.VMEM((nj, K), i32),
            pltpu.SemaphoreType.DMA,
            pltpu.SemaphoreType.DMA,
        ],
    )(functools.partial(_scat_body, ech, nj))


_scat_call = _make_scat(ECH2, NJ2)


def _geom_body(boff, vec_ref, bm_ref, bemb_ref, cut_ref, feat_ref, env_ref):
    v = vec_ref[...]
    vx, vy, vz = v[0], v[1], v[2]
    r = jnp.sqrt(vx * vx + vy * vy + vz * vz + 1e-12)
    inv = 1.0 / (r + 1e-9)
    ux, uy, uz = vx * inv, vy * inv, vz * inv
    pref = np.float32(np.sqrt(2.0 / R_MAX))
    cols = [pref * jnp.sin(np.float32(n * np.pi / R_MAX) * r) * inv
            for n in range(1, NB + 1)]
    s3 = np.float32(np.sqrt(3.0))
    s5 = np.float32(np.sqrt(5.0))
    s15 = np.float32(np.sqrt(15.0))
    cols += [jnp.ones_like(ux), s3 * ux, s3 * uy, s3 * uz,
             s15 * ux * uy, s15 * uy * uz,
             np.float32(0.5) * s5 * (3.0 * uz * uz - 1.0),
             s15 * ux * uz,
             np.float32(0.5) * s15 * (ux * ux - uy * uy)]
    m = bm_ref[...].astype(f32)
    for j in range(16):
        b0 = bemb_ref[0:1, j:j + 1]
        b1 = bemb_ref[1:2, j:j + 1]
        cols.append((1.0 - m) * b0 + m * b1)
    zero = jnp.zeros_like(r)
    cols += [zero] * 7
    feat_ref[...] = jnp.stack(cols, axis=0)
    cut = cut_ref[...]
    x = jnp.clip(r / cut, 0.0, 1.0)
    env = 0.5 * (jnp.cos(np.float32(np.pi) * x) + 1.0) * (r < cut)
    gid = (lax.broadcasted_iota(i32, (PBLK, 128), 0)
           + (pl.program_id(0) + boff) * PBLK) * 128 \
        + lax.broadcasted_iota(i32, (PBLK, 128), 1)
    env_ref[...] = jnp.where(gid < E, env, 0.0)


def _geom_call(vecp, bmp, bemb, cut, t):
    boff = t * (EP2 // PBLK)
    return pl.pallas_call(
        functools.partial(_geom_body, boff),
        grid=(EP2 // PBLK,),
        in_specs=[pl.BlockSpec((4, PBLK, 128),
                               lambda i: (0, i + boff, 0)),
                  pl.BlockSpec((PBLK, 128), lambda i: (i + boff, 0)),
                  pl.BlockSpec((2, 16), lambda i: (0, 0)),
                  pl.BlockSpec((1, 1), lambda i: (0, 0))],
        out_specs=[pl.BlockSpec((40, PBLK, 128), lambda i: (0, i, 0)),
                   pl.BlockSpec((PBLK, 128), lambda i: (i, 0))],
        out_shape=[jax.ShapeDtypeStruct((40, EP2, 128), f32),
                   jax.ShapeDtypeStruct((EP2, 128), f32)],
    )(vecp, bmp, bemb, cut)


def _edge_mlp_body(ft_ref, w1_ref, b1_ref, w2_ref, b2_ref, e_ref):
    h1 = _silu(_mmT(ft_ref[...], w1_ref[...]) + b1_ref[...])
    e_ref[...] = _rms(_mm(h1, w2_ref[...]) + b2_ref[...])


def _node_body(ati_ref, aci_ref, rci_ref, cn_ref, ta_ref, tb_ref, tc_ref,
               w1_ref, b1_ref, w2_ref, b2_ref, ws_ref, wr_ref,
               h_ref, hs_ref, hr_ref):
    i32w = lax.broadcasted_iota(i32, (1, 32), 1)
    i16w = lax.broadcasted_iota(i32, (1, 16), 1)
    emb_a = _mm((ati_ref[...] == i32w).astype(f32), ta_ref[...])
    emb_b = _mm((aci_ref[...] == i16w).astype(f32), tb_ref[...])
    emb_c = _mm((rci_ref[...] == i32w).astype(f32), tc_ref[...])
    x = jnp.concatenate(
        [emb_a, emb_b, emb_c, cn_ref[...],
         jnp.zeros((emb_a.shape[0], 7), f32)], axis=1)
    h1 = _silu(_mm(x, w1_ref[...]) + b1_ref[...])
    h = _rms(_mm(h1, w2_ref[...]) + b2_ref[...])
    h_ref[...] = h
    hs_ref[...] = _mm(h, ws_ref[...])
    hr_ref[...] = _mm(h, wr_ref[...])


def _mmb(a, b):
    return lax.dot_general(a.astype(jnp.bfloat16), b.astype(jnp.bfloat16),
                           (((1,), (0,)), ((), ())),
                           preferred_element_type=f32)


def _msg_body(e_ref, gs_ref, gr_ref, env_ref, w1e_ref, b1_ref, w2m_ref,
              b2m_ref, w2g_ref, b2g_ref, enew_ref, msg_ref):
    e = e_ref[...]
    pre = (_mmb(e, w1e_ref[...]) + gs_ref[...] + gr_ref[...]
           + b1_ref[...])
    s1 = _silu(pre)
    o1 = _mmb(s1, w2m_ref[...]) + b2m_ref[...]
    og = jnp.sum(s1 * w2g_ref[...], axis=-1, keepdims=True) + b2g_ref[...]
    msg = _rms(o1) * jax.nn.sigmoid(og) * env_ref[...]
    enew_ref[...] = e + msg
    msg_ref[...] = msg


def _upd_body(na, h_ref, *refs):
    aggs = refs[:na]
    (w1h_ref, w1a_ref, b1_ref, w2_ref, b2_ref, ws_ref, wr_ref,
     out_ref, hs_ref, hr_ref) = refs[na:]
    h = h_ref[...]
    a = aggs[0][...]
    for r in aggs[1:]:
        a = a + r[...]
    s1 = _silu(_mm(h, w1h_ref[...]) + _mm(a, w1a_ref[...]) + b1_ref[...])
    hn = h + _rms(_mm(s1, w2_ref[...]) + b2_ref[...])
    out_ref[...] = hn
    hs_ref[...] = _mm(hn, ws_ref[...])
    hr_ref[...] = _mm(hn, wr_ref[...])


def _head_body(h_ref, w1_ref, b1_ref, w2_ref, b2_ref, out_ref):
    s1 = _silu(_mm(h_ref[...], w1_ref[...]) + b1_ref[...])
    out_ref[...] = _mm(s1, w2_ref[...]) + b2_ref[...]


def _full(shape):
    return pl.BlockSpec(shape, lambda i: (0,) * len(shape))


def _rows(blk, width):
    return pl.BlockSpec((blk, width), lambda i: (i, 0))


def _edge_mlp_call(ft, w1, b1, w2, b2):
    rows = ft.shape[1]
    return pl.pallas_call(
        _edge_mlp_body,
        grid=(rows // EBLK,),
        in_specs=[pl.BlockSpec((40, EBLK), lambda i: (0, i)),
                  _full((40, LAT)), _full((1, LAT)), _full((LAT, LAT)),
                  _full((1, LAT))],
        out_specs=_rows(EBLK, LAT),
        out_shape=jax.ShapeDtypeStruct((rows, LAT), f32),
    )(ft, w1, b1, w2, b2)


def _node_call(ati, aci, rci, cn, ta, tb, tc, w1, b1, w2, b2, ws, wr):
    return pl.pallas_call(
        _node_body,
        grid=(NG,),
        in_specs=[_rows(NBLK, 1), _rows(NBLK, 1), _rows(NBLK, 1),
                  _rows(NBLK, 1), _full((32, 32)), _full((16, 16)),
                  _full((32, 16)), _full((72, LAT)), _full((1, LAT)),
                  _full((LAT, LAT)), _full((1, LAT)),
                  _full((LAT, LAT)), _full((LAT, LAT))],
        out_specs=[_rows(NBLK, LAT)] * 3,
        out_shape=[jax.ShapeDtypeStruct((N, LAT), f32)] * 3,
    )(ati, aci, rci, cn, ta, tb, tc, w1, b1, w2, b2, ws, wr)


def _msg_call(e, gs, gr, env, w1e, b1, w2m, b2m, w2g, b2g):
    rows = e.shape[0]
    return pl.pallas_call(
        _msg_body,
        grid=(rows // EBLK,),
        in_specs=[_rows(EBLK, LAT), _rows(EBLK, LAT), _rows(EBLK, LAT),
                  _rows(EBLK, 1), _full((LAT, LAT)), _full((1, LAT)),
                  _full((LAT, LAT)), _full((1, LAT)), _full((1, LAT)),
                  _full((1, 1))],
        out_specs=[_rows(EBLK, LAT), _rows(EBLK, LAT)],
        out_shape=[jax.ShapeDtypeStruct((rows, LAT), f32),
                   jax.ShapeDtypeStruct((rows, LAT), f32)],
    )(e, gs, gr, env, w1e, b1, w2m, b2m, w2g, b2g)


def _upd_call(h, aggs, w1h, w1a, b1, w2, b2, ws, wr):
    na = len(aggs)
    return pl.pallas_call(
        functools.partial(_upd_body, na),
        grid=(NG,),
        in_specs=[_rows(NBLK, LAT)] * (1 + na) +
                 [_full((LAT, LAT)), _full((LAT, LAT)), _full((1, LAT)),
                  _full((LAT, LAT)), _full((1, LAT)),
                  _full((LAT, LAT)), _full((LAT, LAT))],
        out_specs=[_rows(NBLK, LAT)] * 3,
        out_shape=[jax.ShapeDtypeStruct((N, LAT), f32)] * 3,
    )(h, *aggs, w1h, w1a, b1, w2, b2, ws, wr)


def _head_call(h, w1, b1, w2, b2):
    return pl.pallas_call(
        _head_body,
        grid=(NG,),
        in_specs=[_rows(NBLK, LAT), _full((LAT, LAT)), _full((1, LAT)),
                  _full((LAT, 8)), _full((1, 8))],
        out_specs=_rows(NBLK, 8),
        out_shape=jax.ShapeDtypeStruct((N, 8), f32),
    )(h, w1, b1, w2, b2)


def kernel(pos, c_noise, atom_type_emb, atom_code_emb, residue_code_emb,
           bond_emb, node_W1, node_b1, node_W2, node_b2, edge_W1, edge_b1,
           edge_W2, edge_b2, msg_W1, msg_b1, msg_W2, msg_b2, upd_W1, upd_b1,
           upd_W2, upd_b2, head_W1, head_b1, head_W2, head_b2,
           atom_type_index, atom_code_index, residue_code_index,
           residue_sequence_index, bond_mask, edge_index,
           effective_radial_cutoff):
    snd = jnp.pad(edge_index[0].astype(i32), (0, EPAD - E))
    rcv = jnp.pad(edge_index[1].astype(i32), (0, EPAD - E))
    cut = jnp.asarray(effective_radial_cutoff, f32).reshape(1, 1)

    pos4 = jnp.pad(pos, ((0, 0), (0, 1))).reshape(-1)
    vecp = _vec_call(pos4, snd, rcv).reshape(4, EP, 128)

    bmp = jnp.pad(bond_mask.astype(i32), (0, EPAD - E)).reshape(EP, 128)
    w1p = jnp.pad(edge_W1, ((0, 7), (0, 0)))
    env, e = [], []
    for t in range(NH):
        feat_t, envp_t = _geom_call(vecp, bmp, bond_emb, cut, t)
        env.append(envp_t.reshape(EH, 1))
        e.append(_edge_mlp_call(feat_t.reshape(40, EH), w1p,
                                edge_b1.reshape(1, LAT), edge_W2,
                                edge_b2.reshape(1, LAT)))

    h, hs, hr = _node_call(
        atom_type_index.reshape(N, 1).astype(i32),
        atom_code_index.reshape(N, 1).astype(i32),
        residue_code_index.reshape(N, 1).astype(i32),
        c_noise.reshape(N, 1),
        jnp.pad(atom_type_emb, ((0, 12), (0, 0))),
        jnp.pad(atom_code_emb, ((0, 6), (0, 0))),
        jnp.pad(residue_code_emb, ((0, 7), (0, 0))),
        jnp.pad(node_W1, ((0, 7), (0, 0))), node_b1.reshape(1, LAT),
        node_W2, node_b2.reshape(1, LAT),
        msg_W1[0][LAT:2 * LAT], msg_W1[0][2 * LAT:])

    snd_h = [snd[t * EH:(t + 1) * EH] for t in range(NH)]
    rcv_h = [rcv[t * EH:(t + 1) * EH] for t in range(NH)]
    rcv3_h = [r.reshape(NW, NJ2, K) for r in rcv_h]
    zero_h = jnp.zeros((N, LAT), f32)

    for s in range(STEPS):
        parts = []
        for t in range(NH):
            gs, gr = _gath_call(hs, hr, snd_h[t], rcv_h[t])
            e[t], msg = _msg_call(
                e[t], gs, gr, env[t], msg_W1[s][:LAT],
                msg_b1[s].reshape(1, LAT),
                msg_W2[s][:, :LAT], msg_b2[s][:LAT].reshape(1, LAT),
                msg_W2[s][:, LAT:].reshape(1, LAT),
                msg_b2[s][LAT:].reshape(1, 1))
            parts.append(_scat_call(msg, rcv3_h[t], zero_h))
        sn = min(s + 1, STEPS - 1)
        aggs = [p[c] for p in parts for c in range(NC)]
        h, hs, hr = _upd_call(
            h, aggs, upd_W1[s][:LAT], upd_W1[s][LAT:],
            upd_b1[s].reshape(1, LAT), upd_W2[s], upd_b2[s].reshape(1, LAT),
            msg_W1[sn][LAT:2 * LAT], msg_W1[sn][2 * LAT:])

    pred = _head_call(
        h, head_W1, head_b1.reshape(1, LAT),
        jnp.pad(head_W2, ((0, 0), (0, 5))),
        jnp.pad(head_b2, (0, 5)).reshape(1, 8))
    return pred[:, :3]


# speedup vs baseline: 1.0368x; 1.0059x over previous
import functools

import jax
import jax.numpy as jnp
import numpy as np
from jax import lax
from jax.experimental import pallas as pl
from jax.experimental.pallas import tpu as pltpu
from jax.experimental.pallas import tpu_sc as plsc

N = 10000
E = 320000
LAT = 128
STEPS = 3
R_MAX = 5.0
NB = 8

NC, NS = 2, 16
NW = NC * NS
K = 80
NJ = 128
ECH = NJ * K
EPAD = NW * ECH
EP = EPAD // 128
PBLK = 256
NROW = 624
NTAIL = N - NS * NROW

NH = 2
EH = EPAD // NH
ECH2 = ECH // NH
NJ2 = NJ // NH
EP2 = EP // NH

EBLK = 4096
EG = EPAD // EBLK
NBLK = 2000
NG = N // NBLK

_SC_MESH = plsc.VectorSubcoreMesh(core_axis_name="c", subcore_axis_name="s")

f32 = jnp.float32
i32 = jnp.int32
bf16 = jnp.bfloat16


def _mm(a, b):
    return lax.dot_general(a, b, (((1,), (0,)), ((), ())),
                           preferred_element_type=f32)


def _mmT(a, b):
    return lax.dot_general(a, b, (((0,), (0,)), ((), ())),
                           preferred_element_type=f32)


def _rms(x):
    return x * lax.rsqrt(jnp.mean(x * x, axis=-1, keepdims=True) + 1e-6)


def _silu(x):
    return x * jax.nn.sigmoid(x)


def _vec_body(pos_hbm, snd_hbm, rcv_hbm, out_hbm, pos_v, snd_v, rcv_v, obuf):
    wid = lax.axis_index("s") * NC + lax.axis_index("c")
    pltpu.sync_copy(pos_hbm, pos_v)
    pltpu.sync_copy(snd_hbm.at[pl.ds(wid * ECH, ECH)], snd_v)
    pltpu.sync_copy(rcv_hbm.at[pl.ds(wid * ECH, ECH)], rcv_v)

    def body(i, carry):
        s4 = snd_v[pl.ds(i * 16, 16)] * 4
        r4 = rcv_v[pl.ds(i * 16, 16)] * 4
        l16 = lax.iota(i32, 16) + i * 16
        for c in range(3):
            ps = plsc.load_gather(pos_v, [s4 + c])
            pr = plsc.load_gather(pos_v, [r4 + c])
            plsc.store_scatter(obuf, [l16 + c * ECH], pr - ps)
        return carry

    lax.fori_loop(0, ECH // 16, body, 0)
    for c in range(3):
        pltpu.sync_copy(obuf.at[pl.ds(c * ECH, ECH)],
                        out_hbm.at[pl.ds(c * EPAD + wid * ECH, ECH)])


_vec_call = functools.partial(
    pl.kernel,
    out_type=jax.ShapeDtypeStruct((4 * EPAD,), f32),
    mesh=_SC_MESH,
    compiler_params=pltpu.CompilerParams(needs_layout_passes=False),
    scratch_types=[
        pltpu.VMEM((N * 4,), f32),
        pltpu.VMEM((ECH,), i32),
        pltpu.VMEM((ECH,), i32),
        pltpu.VMEM((3 * ECH,), f32),
    ],
)(_vec_body)


_NBUF = 4


_LAG = 2


def _gath_body(ech, nj, hs_hbm, hr_hbm, snd_hbm, rcv_hbm, gs_hbm, gr_hbm,
               snd_v, rcv_v,
               a0, a1, a2, a3, b0, b1, b2, b3,
               sa0, sa1, sa2, sa3, sb0, sb1, sb2, sb3,
               wa0, wa1, wa2, wa3, wb0, wb1, wb2, wb3):
    wid = lax.axis_index("s") * NC + lax.axis_index("c")
    pltpu.sync_copy(snd_hbm.at[pl.ds(wid * ech, ech)], snd_v)
    pltpu.sync_copy(rcv_hbm.at[pl.ds(wid * ech, ech)], rcv_v)
    bufa, bufb = [a0, a1, a2, a3], [b0, b1, b2, b3]
    sema, semb = [sa0, sa1, sa2, sa3], [sb0, sb1, sb2, sb3]
    wema, wemb = [wa0, wa1, wa2, wa3], [wb0, wb1, wb2, wb3]

    def issue_g(j, b):
        pltpu.async_copy(hs_hbm.at[snd_v.at[pl.ds(j * K, K)]],
                         bufa[b], sema[b])
        pltpu.async_copy(hr_hbm.at[rcv_v.at[pl.ds(j * K, K)]],
                         bufb[b], semb[b])

    def wait_g(b):
        pltpu.make_async_copy(hs_hbm.at[pl.ds(0, K)], bufa[b],
                              sema[b]).wait()
        pltpu.make_async_copy(hr_hbm.at[pl.ds(0, K)], bufb[b],
                              semb[b]).wait()

    def issue_w(j, b):
        base = wid * ech + j * K
        pltpu.async_copy(bufa[b], gs_hbm.at[pl.ds(base, K)], wema[b])
        pltpu.async_copy(bufb[b], gr_hbm.at[pl.ds(base, K)], wemb[b])

    def wait_w(b):
        pltpu.make_async_copy(bufa[b], gs_hbm.at[pl.ds(0, K)],
                              wema[b]).wait()
        pltpu.make_async_copy(bufb[b], gr_hbm.at[pl.ds(0, K)],
                              wemb[b]).wait()

    def body(jj, carry):
        for b in range(_NBUF):
            j = jj * _NBUF + b

            @pl.when(jj > 0)
            def _():
                wait_w(b)

            issue_g(j, b)
            bd = (b - _LAG) % _NBUF

            @pl.when(j >= _LAG)
            def _():
                wait_g(bd)
                issue_w(j - _LAG, bd)

        return carry

    lax.fori_loop(0, nj // _NBUF, body, 0)
    for t in range(_LAG):
        j = nj - _LAG + t
        wait_g(j % _NBUF)
        issue_w(j, j % _NBUF)
    for b in range(_NBUF):
        wait_w(b)


def _make_gath(ech, nj, rows):
    return functools.partial(
        pl.kernel,
        out_type=(jax.ShapeDtypeStruct((rows, LAT), f32),
                  jax.ShapeDtypeStruct((rows, LAT), f32)),
        mesh=_SC_MESH,
        scratch_types=[
            pltpu.VMEM((ech,), i32),
            pltpu.VMEM((ech,), i32),
        ] + [pltpu.VMEM((K, LAT), f32)] * (2 * _NBUF)
          + [pltpu.SemaphoreType.DMA] * (4 * _NBUF),
    )(functools.partial(_gath_body, ech, nj))


_gath_call = _make_gath(ECH2, NJ2, EH)


def _scat_body(ech, nj, msg_hbm, rcv3_hbm, out_hbm, shared,
               m0, m1, rbuf, zbuf, sl0, sl1):
    cid = lax.axis_index("c")
    sid = lax.axis_index("s")
    wid = sid * NC + cid
    pltpu.sync_copy(rcv3_hbm.at[wid], rbuf)

    def zfill(r, carry):
        for c in range(LAT // 16):
            zbuf[r, pl.ds(c * 16, 16)] = jnp.zeros((16,), f32)
        return carry

    lax.fori_loop(0, K, zfill, 0)
    for k in range(NROW // K):
        pltpu.sync_copy(zbuf, shared.at[pl.ds(sid * NROW + k * K, K)])
    rem = NROW - (NROW // K) * K
    pltpu.sync_copy(zbuf.at[pl.ds(0, rem)],
                    shared.at[pl.ds(sid * NROW + NROW - rem, rem)])

    @pl.when(sid == NS - 1)
    def _():
        pltpu.sync_copy(zbuf.at[pl.ds(0, NTAIL)],
                        shared.at[pl.ds(NS * NROW, NTAIL)])

    plsc.subcore_barrier()
    bufm = [m0, m1]
    seml = [sl0, sl1]

    def issue(j, b):
        pltpu.async_copy(msg_hbm.at[pl.ds(wid * ech + j * K, K)],
                         bufm[b], seml[b])

    def drain(j, b):
        pltpu.make_async_copy(msg_hbm.at[pl.ds(0, K)], bufm[b],
                              seml[b]).wait()
        pltpu.sync_copy(bufm[b], shared.at[rbuf.at[j]], add=True)

    issue(0, 0)
    issue(1, 1)

    def body(jj, carry):
        for b in range(2):
            j = jj * 2 + b
            drain(j, b)
            issue(j + 2, b)
        return carry

    lax.fori_loop(0, nj // 2 - 1, body, 0)
    for b in range(2):
        drain(nj - 2 + b, b)
    plsc.subcore_barrier()
    pltpu.sync_copy(shared.at[pl.ds(sid * NROW, NROW)],
                    out_hbm.at[cid, pl.ds(sid * NROW, NROW)])

    @pl.when(sid == NS - 1)
    def _():
        pltpu.sync_copy(shared.at[pl.ds(NS * NROW, NTAIL)],
                        out_hbm.at[cid, pl.ds(NS * NROW, NTAIL)])


def _make_scat(ech, nj):
    return functools.partial(
        pl.kernel,
        out_type=jax.ShapeDtypeStruct((NC, N, LAT), f32),
        mesh=_SC_MESH,
        scratch_types=[
            pltpu.VMEM_SHARED((N, LAT), f32),
            pltpu.VMEM((K, LAT), f32),
            pltpu.VMEM((K, LAT), f32),
            pltpu.VMEM((nj, K), i32),
            pltpu.VMEM((K, LAT), f32),
            pltpu.SemaphoreType.DMA,
            pltpu.SemaphoreType.DMA,
        ],
    )(functools.partial(_scat_body, ech, nj))


_scat_call = _make_scat(ECH2, NJ2)


def _geom_body(boff, vec_ref, bm_ref, bemb_ref, cut_ref, feat_ref, env_ref):
    v = vec_ref[...]
    vx, vy, vz = v[0], v[1], v[2]
    r = jnp.sqrt(vx * vx + vy * vy + vz * vz + 1e-12)
    inv = 1.0 / (r + 1e-9)
    ux, uy, uz = vx * inv, vy * inv, vz * inv
    pref = np.float32(np.sqrt(2.0 / R_MAX))
    cols = [pref * jnp.sin(np.float32(n * np.pi / R_MAX) * r) * inv
            for n in range(1, NB + 1)]
    s3 = np.float32(np.sqrt(3.0))
    s5 = np.float32(np.sqrt(5.0))
    s15 = np.float32(np.sqrt(15.0))
    cols += [jnp.ones_like(ux), s3 * ux, s3 * uy, s3 * uz,
             s15 * ux * uy, s15 * uy * uz,
             np.float32(0.5) * s5 * (3.0 * uz * uz - 1.0),
             s15 * ux * uz,
             np.float32(0.5) * s15 * (ux * ux - uy * uy)]
    m = bm_ref[...].astype(f32)
    for j in range(16):
        b0 = bemb_ref[0:1, j:j + 1]
        b1 = bemb_ref[1:2, j:j + 1]
        cols.append((1.0 - m) * b0 + m * b1)
    zero = jnp.zeros_like(r)
    cols += [zero] * 7
    feat_ref[...] = jnp.stack(cols, axis=0)
    cut = cut_ref[...]
    x = jnp.clip(r / cut, 0.0, 1.0)
    env = 0.5 * (jnp.cos(np.float32(np.pi) * x) + 1.0) * (r < cut)
    gid = (lax.broadcasted_iota(i32, (PBLK, 128), 0)
           + (pl.program_id(0) + boff) * PBLK) * 128 \
        + lax.broadcasted_iota(i32, (PBLK, 128), 1)
    env_ref[...] = jnp.where(gid < E, env, 0.0)


def _geom_call(vecp, bmp, bemb, cut, t):
    boff = t * (EP2 // PBLK)
    return pl.pallas_call(
        functools.partial(_geom_body, boff),
        grid=(EP2 // PBLK,),
        in_specs=[pl.BlockSpec((4, PBLK, 128),
                               lambda i: (0, i + boff, 0)),
                  pl.BlockSpec((PBLK, 128), lambda i: (i + boff, 0)),
                  pl.BlockSpec((2, 16), lambda i: (0, 0)),
                  pl.BlockSpec((1, 1), lambda i: (0, 0))],
        out_specs=[pl.BlockSpec((40, PBLK, 128), lambda i: (0, i, 0)),
                   pl.BlockSpec((PBLK, 128), lambda i: (i, 0))],
        out_shape=[jax.ShapeDtypeStruct((40, EP2, 128), f32),
                   jax.ShapeDtypeStruct((EP2, 128), f32)],
    )(vecp, bmp, bemb, cut)


def _edge_mlp_body(ft_ref, w1_ref, b1_ref, w2_ref, b2_ref, e_ref):
    h1 = _silu(_mmT(ft_ref[...], w1_ref[...]) + b1_ref[...])
    e_ref[...] = _rms(_mm(h1, w2_ref[...]) + b2_ref[...])


def _node_body(ati_ref, aci_ref, rci_ref, cn_ref, ta_ref, tb_ref, tc_ref,
               w1_ref, b1_ref, w2_ref, b2_ref, ws_ref, wr_ref,
               h_ref, hs_ref, hr_ref):
    i32w = lax.broadcasted_iota(i32, (1, 32), 1)
    i16w = lax.broadcasted_iota(i32, (1, 16), 1)
    emb_a = _mm((ati_ref[...] == i32w).astype(f32), ta_ref[...])
    emb_b = _mm((aci_ref[...] == i16w).astype(f32), tb_ref[...])
    emb_c = _mm((rci_ref[...] == i32w).astype(f32), tc_ref[...])
    x = jnp.concatenate(
        [emb_a, emb_b, emb_c, cn_ref[...],
         jnp.zeros((emb_a.shape[0], 7), f32)], axis=1)
    h1 = _silu(_mm(x, w1_ref[...]) + b1_ref[...])
    h = _rms(_mm(h1, w2_ref[...]) + b2_ref[...])
    h_ref[...] = h
    hs_ref[...] = _mm(h, ws_ref[...])
    hr_ref[...] = _mm(h, wr_ref[...])


def _mmb(a, b):
    return lax.dot_general(a.astype(jnp.bfloat16), b.astype(jnp.bfloat16),
                           (((1,), (0,)), ((), ())),
                           preferred_element_type=f32)


def _msg_body(e_ref, gs_ref, gr_ref, env_ref, w1e_ref, b1_ref, w2m_ref,
              b2m_ref, w2g_ref, b2g_ref, enew_ref, msg_ref):
    e = e_ref[...]
    pre = (_mmb(e, w1e_ref[...]) + gs_ref[...] + gr_ref[...]
           + b1_ref[...])
    s1 = _silu(pre)
    o1 = _mmb(s1, w2m_ref[...]) + b2m_ref[...]
    og = jnp.sum(s1 * w2g_ref[...], axis=-1, keepdims=True) + b2g_ref[...]
    msg = _rms(o1) * jax.nn.sigmoid(og) * env_ref[...]
    enew_ref[...] = e + msg
    msg_ref[...] = msg


def _upd_body(na, h_ref, *refs):
    aggs = refs[:na]
    (w1h_ref, w1a_ref, b1_ref, w2_ref, b2_ref, ws_ref, wr_ref,
     out_ref, hs_ref, hr_ref) = refs[na:]
    h = h_ref[...]
    a = aggs[0][...]
    for r in aggs[1:]:
        a = a + r[...]
    s1 = _silu(_mm(h, w1h_ref[...]) + _mm(a, w1a_ref[...]) + b1_ref[...])
    hn = h + _rms(_mm(s1, w2_ref[...]) + b2_ref[...])
    out_ref[...] = hn
    hs_ref[...] = _mm(hn, ws_ref[...])
    hr_ref[...] = _mm(hn, wr_ref[...])


def _head_body(h_ref, w1_ref, b1_ref, w2_ref, b2_ref, out_ref):
    s1 = _silu(_mm(h_ref[...], w1_ref[...]) + b1_ref[...])
    out_ref[...] = _mm(s1, w2_ref[...]) + b2_ref[...]


def _full(shape):
    return pl.BlockSpec(shape, lambda i: (0,) * len(shape))


def _rows(blk, width):
    return pl.BlockSpec((blk, width), lambda i: (i, 0))


def _edge_mlp_call(ft, w1, b1, w2, b2):
    rows = ft.shape[1]
    return pl.pallas_call(
        _edge_mlp_body,
        grid=(rows // EBLK,),
        in_specs=[pl.BlockSpec((40, EBLK), lambda i: (0, i)),
                  _full((40, LAT)), _full((1, LAT)), _full((LAT, LAT)),
                  _full((1, LAT))],
        out_specs=_rows(EBLK, LAT),
        out_shape=jax.ShapeDtypeStruct((rows, LAT), f32),
    )(ft, w1, b1, w2, b2)


def _node_call(ati, aci, rci, cn, ta, tb, tc, w1, b1, w2, b2, ws, wr):
    return pl.pallas_call(
        _node_body,
        grid=(NG,),
        in_specs=[_rows(NBLK, 1), _rows(NBLK, 1), _rows(NBLK, 1),
                  _rows(NBLK, 1), _full((32, 32)), _full((16, 16)),
                  _full((32, 16)), _full((72, LAT)), _full((1, LAT)),
                  _full((LAT, LAT)), _full((1, LAT)),
                  _full((LAT, LAT)), _full((LAT, LAT))],
        out_specs=[_rows(NBLK, LAT)] * 3,
        out_shape=[jax.ShapeDtypeStruct((N, LAT), f32)] * 3,
    )(ati, aci, rci, cn, ta, tb, tc, w1, b1, w2, b2, ws, wr)


def _msg_call(e, gs, gr, env, w1e, b1, w2m, b2m, w2g, b2g):
    rows = e.shape[0]
    return pl.pallas_call(
        _msg_body,
        grid=(rows // EBLK,),
        in_specs=[_rows(EBLK, LAT), _rows(EBLK, LAT), _rows(EBLK, LAT),
                  _rows(EBLK, 1), _full((LAT, LAT)), _full((1, LAT)),
                  _full((LAT, LAT)), _full((1, LAT)), _full((1, LAT)),
                  _full((1, 1))],
        out_specs=[_rows(EBLK, LAT), _rows(EBLK, LAT)],
        out_shape=[jax.ShapeDtypeStruct((rows, LAT), f32),
                   jax.ShapeDtypeStruct((rows, LAT), f32)],
    )(e, gs, gr, env, w1e, b1, w2m, b2m, w2g, b2g)


def _upd_call(h, aggs, w1h, w1a, b1, w2, b2, ws, wr):
    na = len(aggs)
    return pl.pallas_call(
        functools.partial(_upd_body, na),
        grid=(NG,),
        in_specs=[_rows(NBLK, LAT)] * (1 + na) +
                 [_full((LAT, LAT)), _full((LAT, LAT)), _full((1, LAT)),
                  _full((LAT, LAT)), _full((1, LAT)),
                  _full((LAT, LAT)), _full((LAT, LAT))],
        out_specs=[_rows(NBLK, LAT)] * 3,
        out_shape=[jax.ShapeDtypeStruct((N, LAT), f32)] * 3,
    )(h, *aggs, w1h, w1a, b1, w2, b2, ws, wr)


def _head_call(h, w1, b1, w2, b2):
    return pl.pallas_call(
        _head_body,
        grid=(NG,),
        in_specs=[_rows(NBLK, LAT), _full((LAT, LAT)), _full((1, LAT)),
                  _full((LAT, 8)), _full((1, 8))],
        out_specs=_rows(NBLK, 8),
        out_shape=jax.ShapeDtypeStruct((N, 8), f32),
    )(h, w1, b1, w2, b2)


def kernel(pos, c_noise, atom_type_emb, atom_code_emb, residue_code_emb,
           bond_emb, node_W1, node_b1, node_W2, node_b2, edge_W1, edge_b1,
           edge_W2, edge_b2, msg_W1, msg_b1, msg_W2, msg_b2, upd_W1, upd_b1,
           upd_W2, upd_b2, head_W1, head_b1, head_W2, head_b2,
           atom_type_index, atom_code_index, residue_code_index,
           residue_sequence_index, bond_mask, edge_index,
           effective_radial_cutoff):
    snd = jnp.pad(edge_index[0].astype(i32), (0, EPAD - E))
    rcv = jnp.pad(edge_index[1].astype(i32), (0, EPAD - E))
    cut = jnp.asarray(effective_radial_cutoff, f32).reshape(1, 1)

    pos4 = jnp.pad(pos, ((0, 0), (0, 1))).reshape(-1)
    vecp = _vec_call(pos4, snd, rcv).reshape(4, EP, 128)

    bmp = jnp.pad(bond_mask.astype(i32), (0, EPAD - E)).reshape(EP, 128)
    w1p = jnp.pad(edge_W1, ((0, 7), (0, 0)))
    env, e = [], []
    for t in range(NH):
        feat_t, envp_t = _geom_call(vecp, bmp, bond_emb, cut, t)
        env.append(envp_t.reshape(EH, 1))
        e.append(_edge_mlp_call(feat_t.reshape(40, EH), w1p,
                                edge_b1.reshape(1, LAT), edge_W2,
                                edge_b2.reshape(1, LAT)))

    h, hs, hr = _node_call(
        atom_type_index.reshape(N, 1).astype(i32),
        atom_code_index.reshape(N, 1).astype(i32),
        residue_code_index.reshape(N, 1).astype(i32),
        c_noise.reshape(N, 1),
        jnp.pad(atom_type_emb, ((0, 12), (0, 0))),
        jnp.pad(atom_code_emb, ((0, 6), (0, 0))),
        jnp.pad(residue_code_emb, ((0, 7), (0, 0))),
        jnp.pad(node_W1, ((0, 7), (0, 0))), node_b1.reshape(1, LAT),
        node_W2, node_b2.reshape(1, LAT),
        msg_W1[0][LAT:2 * LAT], msg_W1[0][2 * LAT:])

    snd_h = [snd[t * EH:(t + 1) * EH] for t in range(NH)]
    rcv_h = [rcv[t * EH:(t + 1) * EH] for t in range(NH)]
    rcv3_h = [r.reshape(NW, NJ2, K) for r in rcv_h]

    for s in range(STEPS):
        parts = []
        for t in range(NH):
            gs, gr = _gath_call(hs, hr, snd_h[t], rcv_h[t])
            e[t], msg = _msg_call(
                e[t], gs, gr, env[t], msg_W1[s][:LAT],
                msg_b1[s].reshape(1, LAT),
                msg_W2[s][:, :LAT], msg_b2[s][:LAT].reshape(1, LAT),
                msg_W2[s][:, LAT:].reshape(1, LAT),
                msg_b2[s][LAT:].reshape(1, 1))
            parts.append(_scat_call(msg, rcv3_h[t]))
        sn = min(s + 1, STEPS - 1)
        aggs = [p[c] for p in parts for c in range(NC)]
        h, hs, hr = _upd_call(
            h, aggs, upd_W1[s][:LAT], upd_W1[s][LAT:],
            upd_b1[s].reshape(1, LAT), upd_W2[s], upd_b2[s].reshape(1, LAT),
            msg_W1[sn][LAT:2 * LAT], msg_W1[sn][2 * LAT:])

    pred = _head_call(
        h, head_W1, head_b1.reshape(1, LAT),
        jnp.pad(head_W2, ((0, 0), (0, 5))),
        jnp.pad(head_b2, (0, 5)).reshape(1, 8))
    return pred[:, :3]
